# baseline (speedup 1.0000x reference)
import functools

import jax
import jax.numpy as jnp
from jax.experimental import pallas as pl
from jax.experimental.pallas import tpu as pltpu

_K = 16
_BLOCK_N = 256
_R = 6
_LANES = 128


def _topk_block(dist_ref, m):
    bn = dist_ref.shape[0]
    ngroups = m // _LANES
    inf = jnp.float32(jnp.inf)
    lane = jax.lax.broadcasted_iota(jnp.int32, (bn, _LANES), 1)
    vals = [jnp.full((bn, _LANES), inf, jnp.float32) for _ in range(_R)]
    cols = [jnp.full((bn, _LANES), m, jnp.int32) for _ in range(_R)]
    for g in range(ngroups):
        t = dist_ref[:, g * _LANES:(g + 1) * _LANES]
        tc = lane + (g * _LANES)
        for j in range(_R):
            c = t < vals[j]
            vals[j], t = jnp.where(c, t, vals[j]), jnp.where(c, vals[j], t)
            cols[j], tc = jnp.where(c, tc, cols[j]), jnp.where(c, cols[j], tc)
    kcol = jax.lax.broadcasted_iota(jnp.int32, (bn, _K), 1)
    out = jnp.zeros((bn, _K), jnp.int32)
    for k in range(_K):
        gv = jnp.min(vals[0], axis=1, keepdims=True)
        eq = vals[0] == gv
        win = jnp.min(jnp.where(eq, cols[0], m), axis=1)
        out = jnp.where(kcol == k, win[:, None], out)
        pop = eq & (cols[0] == win[:, None])
        for j in range(_R - 1):
            vals[j] = jnp.where(pop, vals[j + 1], vals[j])
            cols[j] = jnp.where(pop, cols[j + 1], cols[j])
        vals[_R - 1] = jnp.where(pop, inf, vals[_R - 1])
        cols[_R - 1] = jnp.where(pop, m, cols[_R - 1])
    return out


def _knn_body(nblocks, xt_ref, bt_ref, y2_ref, out_ref, sa_ref, sb_ref):
    s = pl.program_id(0)
    bt = bt_ref[0]
    y2 = y2_ref[0]
    m = bt.shape[1]
    bn = _BLOCK_N
    last = nblocks - 1
    blocks_per_batch = nblocks // 2

    def mm(block_off):
        a = xt_ref[0, pl.ds(block_off, bn), :]
        x2 = jnp.sum(a * a, axis=1, keepdims=True)
        inner = jax.lax.dot_general(
            a, bt, (((1,), (0,)), ((), ())),
            preferred_element_type=jnp.float32)
        return (x2 + (-2.0) * inner) + y2

    def store_out(block_idx, res):
        bb = block_idx // blocks_per_batch
        ro = (block_idx % blocks_per_batch) * bn
        out_ref[bb, pl.ds(ro, bn), :] = res

    even_blk = jnp.minimum(2 * s, last)
    a_off = jnp.where(2 * s > last, bn, 0)
    sa_ref[...] = mm(a_off)
    store_out(jnp.maximum(2 * s - 1, 0), _topk_block(sb_ref, m))
    sb_ref[...] = mm(bn)
    store_out(even_blk, _topk_block(sa_ref, m))


def _normalize(v, axis):
    n = jnp.sqrt(jnp.sum(v * v, axis=axis, keepdims=True))
    return v / jnp.maximum(n, 1e-12)


@jax.jit
def kernel(x, y):
    xn = _normalize(x, 1)[..., 0]
    yn = _normalize(y, 1)[..., 0]
    xt = jnp.transpose(xn, (0, 2, 1))
    b, n, c = xt.shape
    m = yn.shape[2]
    y2 = jnp.sum(yn * yn, axis=1, keepdims=True)

    nblocks = (b * n) // _BLOCK_N
    dsteps = nblocks // 2
    grid = (dsteps + 1,)
    two_bn = 2 * _BLOCK_N
    dbl_per_batch = n // two_bn

    def xt_map(s):
        g = jnp.minimum(s, dsteps - 1)
        return (g // dbl_per_batch, g % dbl_per_batch, 0)

    def b_map(s):
        g = jnp.minimum(s, dsteps - 1)
        return (g // dbl_per_batch, 0, 0)

    nn_idx = pl.pallas_call(
        functools.partial(_knn_body, nblocks),
        grid=grid,
        in_specs=[
            pl.BlockSpec((1, two_bn, c), xt_map),
            pl.BlockSpec((1, c, m), b_map),
            pl.BlockSpec((1, 1, m), b_map),
        ],
        out_specs=pl.BlockSpec((b, n, _K), lambda s: (0, 0, 0)),
        out_shape=jax.ShapeDtypeStruct((b, n, _K), jnp.int32),
        scratch_shapes=[
            pltpu.VMEM((_BLOCK_N, m), jnp.float32),
            pltpu.VMEM((_BLOCK_N, m), jnp.float32),
        ],
    )(xt, yn, y2)

    center_idx = jnp.broadcast_to(
        jnp.arange(n, dtype=nn_idx.dtype)[None, :, None], (b, n, _K))
    return jnp.stack((nn_idx, center_idx), axis=0)


# speedup vs baseline: 1.1688x; 1.1688x over previous
import jax
import jax.numpy as jnp
from jax.experimental import pallas as pl

_K = 16
_BLOCK_N = 256
_R = 6
_LANES = 128
_CHUNK = 256
_BIAS = 0x3F000000
_MAXKEY = 0x7FFFFFFF


def _knn_body(a_ref, bt_ref, x2_ref, y2_ref, out_ref):
    a = a_ref[0]
    x2 = x2_ref[0]
    y2 = y2_ref[0]
    bn = a.shape[0]
    m = bt_ref.shape[2]
    nchunks = m // _CHUNK
    gpc = _CHUNK // _LANES

    lane = jax.lax.broadcasted_iota(jnp.int32, (bn, _LANES), 1)
    keys = [jnp.full((bn, _LANES), _MAXKEY, jnp.int32) for _ in range(_R)]
    for ci in range(nchunks):
        btc = bt_ref[0, :, ci * _CHUNK:(ci + 1) * _CHUNK]
        inner = jax.lax.dot_general(
            a, btc, (((1,), (0,)), ((), ())),
            preferred_element_type=jnp.float32)
        d = (x2 + (-2.0) * inner) + y2[:, ci * _CHUNK:(ci + 1) * _CHUNK]
        bits = jax.lax.bitcast_convert_type(d, jnp.int32)
        for s in range(gpc):
            g = ci * gpc + s
            t = (jnp.maximum(bits[:, s * _LANES:(s + 1) * _LANES] - _BIAS, 0)
                 << 5) | g
            for j in range(_R):
                c = t < keys[j]
                keys[j], t = (jnp.where(c, t, keys[j]),
                              jnp.where(c, keys[j], t))

    kcol = jax.lax.broadcasted_iota(jnp.int32, (bn, _K), 1)
    out = jnp.zeros((bn, _K), jnp.int32)
    for k in range(_K):
        gv = jnp.min(keys[0], axis=1, keepdims=True)
        eq = keys[0] == gv
        lane_w = jnp.min(jnp.where(eq, lane, _LANES), axis=1)
        col = ((gv[:, 0] & 31) << 7) | lane_w
        out = jnp.where(kcol == k, col[:, None], out)
        pop = eq & (lane == lane_w[:, None])
        for j in range(_R - 1):
            keys[j] = jnp.where(pop, keys[j + 1], keys[j])
        keys[_R - 1] = jnp.where(pop, _MAXKEY, keys[_R - 1])
    out_ref[0] = out


def _normalize(v, axis):
    n = jnp.sqrt(jnp.sum(v * v, axis=axis, keepdims=True))
    return v / jnp.maximum(n, 1e-12)


@jax.jit
def kernel(x, y):
    xn = _normalize(x, 1)[..., 0]
    yn = _normalize(y, 1)[..., 0]
    xt = jnp.transpose(xn, (0, 2, 1))
    b, n, c = xt.shape
    m = yn.shape[2]
    x2 = jnp.sum(xt * xt, axis=-1, keepdims=True)
    y2 = jnp.sum(yn * yn, axis=1, keepdims=True)

    grid = (b, n // _BLOCK_N)
    nn_idx = pl.pallas_call(
        _knn_body,
        grid=grid,
        in_specs=[
            pl.BlockSpec((1, _BLOCK_N, c), lambda i, j: (i, j, 0)),
            pl.BlockSpec((1, c, m), lambda i, j: (i, 0, 0)),
            pl.BlockSpec((1, _BLOCK_N, 1), lambda i, j: (i, j, 0)),
            pl.BlockSpec((1, 1, m), lambda i, j: (i, 0, 0)),
        ],
        out_specs=pl.BlockSpec((1, _BLOCK_N, _K), lambda i, j: (i, j, 0)),
        out_shape=jax.ShapeDtypeStruct((b, n, _K), jnp.int32),
    )(xt, yn, x2, y2)

    center_idx = jnp.broadcast_to(
        jnp.arange(n, dtype=nn_idx.dtype)[None, :, None], (b, n, _K))
    return jnp.stack((nn_idx, center_idx), axis=0)


# speedup vs baseline: 1.2021x; 1.0285x over previous
import jax
import jax.numpy as jnp
from jax.experimental import pallas as pl

_K = 16
_BLOCK_N = 256
_R = 6
_LANES = 128
_CHUNK = 256
_BIAS = 0x3F000000
_MAXKEY = 0x7FFFFFFF


def _knn_body(a_ref, bt_ref, x2_ref, y2_ref, out_ref):
    a = a_ref[0]
    x2 = x2_ref[0]
    y2 = y2_ref[0]
    bn = a.shape[0]
    m = bt_ref.shape[2]
    nchunks = m // _CHUNK
    gpc = _CHUNK // _LANES

    lane = jax.lax.broadcasted_iota(jnp.int32, (bn, _LANES), 1)
    keys = [jnp.full((bn, _LANES), _MAXKEY, jnp.int32) for _ in range(_R)]

    def ce(arr, i, j):
        lo = jnp.minimum(arr[i], arr[j])
        arr[j] = jnp.maximum(arr[i], arr[j])
        arr[i] = lo

    def merge_batch(batch):
        for (i, j) in ((0, 1), (2, 3), (4, 5), (6, 7),
                       (0, 2), (1, 3), (4, 6), (5, 7),
                       (1, 2), (5, 6),
                       (0, 4), (1, 5), (2, 6), (3, 7),
                       (2, 4), (3, 5),
                       (1, 2), (3, 4), (5, 6)):
            ce(batch, i, j)
        for j in range(_R):
            keys[j] = jnp.minimum(keys[j], batch[_R - 1 - j])
        for r in range(_R):
            for i in range(r & 1, _R - 1, 2):
                ce(keys, i, i + 1)

    maxslab = jnp.full((bn, _LANES), _MAXKEY, jnp.int32)
    batch = []
    for ci in range(nchunks):
        btc = bt_ref[0, :, ci * _CHUNK:(ci + 1) * _CHUNK]
        inner = jax.lax.dot_general(
            a, btc, (((1,), (0,)), ((), ())),
            preferred_element_type=jnp.float32)
        d = (x2 + (-2.0) * inner) + y2[:, ci * _CHUNK:(ci + 1) * _CHUNK]
        bits = jax.lax.bitcast_convert_type(d, jnp.int32)
        for s in range(gpc):
            g = ci * gpc + s
            batch.append(
                (jnp.maximum(bits[:, s * _LANES:(s + 1) * _LANES], _BIAS)
                 << 5) | g)
        if len(batch) == 8:
            merge_batch(batch)
            batch = []
    if batch:
        merge_batch(batch + [maxslab] * (8 - len(batch)))

    kcol = jax.lax.broadcasted_iota(jnp.int32, (bn, _K), 1)
    out = jnp.zeros((bn, _K), jnp.int32)
    for k in range(_K):
        gv = jnp.min(keys[0], axis=1, keepdims=True)
        eq = keys[0] == gv
        lane_w = jnp.min(jnp.where(eq, lane, _LANES), axis=1)
        col = ((gv[:, 0] & 31) << 7) | lane_w
        out = jnp.where(kcol == k, col[:, None], out)
        pop = eq & (lane == lane_w[:, None])
        for j in range(_R - 1):
            keys[j] = jnp.where(pop, keys[j + 1], keys[j])
        keys[_R - 1] = jnp.where(pop, _MAXKEY, keys[_R - 1])
    out_ref[0] = out


def _normalize(v, axis):
    n = jnp.sqrt(jnp.sum(v * v, axis=axis, keepdims=True))
    return v / jnp.maximum(n, 1e-12)


@jax.jit
def kernel(x, y):
    xn = _normalize(x, 1)[..., 0]
    yn = _normalize(y, 1)[..., 0]
    xt = jnp.transpose(xn, (0, 2, 1))
    b, n, c = xt.shape
    m = yn.shape[2]
    x2 = jnp.sum(xt * xt, axis=-1, keepdims=True)
    y2 = jnp.sum(yn * yn, axis=1, keepdims=True)

    grid = (b, n // _BLOCK_N)
    nn_idx = pl.pallas_call(
        _knn_body,
        grid=grid,
        in_specs=[
            pl.BlockSpec((1, _BLOCK_N, c), lambda i, j: (i, j, 0)),
            pl.BlockSpec((1, c, m), lambda i, j: (i, 0, 0)),
            pl.BlockSpec((1, _BLOCK_N, 1), lambda i, j: (i, j, 0)),
            pl.BlockSpec((1, 1, m), lambda i, j: (i, 0, 0)),
        ],
        out_specs=pl.BlockSpec((1, _BLOCK_N, _K), lambda i, j: (i, j, 0)),
        out_shape=jax.ShapeDtypeStruct((b, n, _K), jnp.int32),
    )(xt, yn, x2, y2)

    center_idx = jnp.broadcast_to(
        jnp.arange(n, dtype=nn_idx.dtype)[None, :, None], (b, n, _K))
    return jnp.stack((nn_idx, center_idx), axis=0)
